# baseline (speedup 1.0000x reference)
import jax
import jax.numpy as jnp
from jax.experimental import pallas as pl
from jax.experimental.pallas import tpu as pltpu

_ROWS2 = 50000
_BLK = 5000


def _copy_kernel(u_in, i_in, u_out, i_out):
    u_out[...] = u_in[...]
    i_out[...] = i_in[...]


def kernel(user_weight, item_weight):
    u2 = user_weight.reshape(_ROWS2, 128)
    i2 = item_weight.reshape(_ROWS2, 128)
    grid = _ROWS2 // _BLK
    spec = pl.BlockSpec((_BLK, 128), lambda n: (n, 0))
    ou, oi = pl.pallas_call(
        _copy_kernel,
        grid=(grid,),
        out_shape=(
            jax.ShapeDtypeStruct(u2.shape, u2.dtype),
            jax.ShapeDtypeStruct(i2.shape, i2.dtype),
        ),
        in_specs=[spec, spec],
        out_specs=(spec, spec),
    )(u2, i2)
    return ou.reshape(100000, 64), oi.reshape(100000, 64)


# speedup vs baseline: 1.1956x; 1.1956x over previous
import functools

import jax
import jax.numpy as jnp
from jax import lax
from jax.experimental import pallas as pl
from jax.experimental.pallas import tpu as pltpu
from jax.experimental.pallas import tpu_sc as plsc

_ROWS = 100000
_EMBED = 64
_NW = 32
_CHUNK = 400
_NCHUNK = _ROWS // _CHUNK
_ROUNDS = -(-_NCHUNK // _NW)


@functools.partial(
    pl.kernel,
    out_type=(
        jax.ShapeDtypeStruct((_ROWS, _EMBED), jnp.float32),
        jax.ShapeDtypeStruct((_ROWS, _EMBED), jnp.float32),
    ),
    mesh=plsc.VectorSubcoreMesh(core_axis_name="c", subcore_axis_name="s"),
    scratch_types=[
        pltpu.VMEM((_CHUNK, _EMBED), jnp.float32),
        pltpu.VMEM((_CHUNK, _EMBED), jnp.float32),
        pltpu.SemaphoreType.DMA,
        pltpu.SemaphoreType.DMA,
    ],
)
def _sc_copy(u_in, i_in, u_out, i_out, buf_u, buf_i, sem_u, sem_i):
    wid = lax.axis_index("s") * 2 + lax.axis_index("c")

    for r in range(_ROUNDS):
        c = r * _NW + wid
        off = c * _CHUNK

        def _round(off=off):
            sl = pl.ds(off, _CHUNK)
            gu = pltpu.async_copy(u_in.at[sl], buf_u, sem_u)
            gi = pltpu.async_copy(i_in.at[sl], buf_i, sem_i)
            gu.wait()
            su = pltpu.async_copy(buf_u, u_out.at[sl], sem_u)
            gi.wait()
            si = pltpu.async_copy(buf_i, i_out.at[sl], sem_i)
            su.wait()
            si.wait()

        if (r + 1) * _NW <= _NCHUNK:
            _round()
        else:
            pl.when(c < _NCHUNK)(_round)


def kernel(user_weight, item_weight):
    return _sc_copy(user_weight, item_weight)


# speedup vs baseline: 1.3353x; 1.1169x over previous
import jax
import jax.numpy as jnp
from jax.experimental import pallas as pl
from jax.experimental.pallas import tpu as pltpu

_ROWS = 100000
_EMBED = 64
_CHUNK = 1000
_NCHUNK = _ROWS // _CHUNK
_LANES_PER_TABLE = 8


def _make_lanes():
    lanes = []
    for t in range(2):
        for l in range(_LANES_PER_TABLE):
            offs = [c * _CHUNK for c in range(l, _NCHUNK, _LANES_PER_TABLE)]
            lanes.append((t, offs))
    return lanes


_LANES = _make_lanes()
_MAX_ITERS = max(len(offs) for _, offs in _LANES)


def _copy_kernel(u_in, i_in, u_out, i_out, *scratch):
    bufs = scratch[: len(_LANES)]
    sems = scratch[len(_LANES):]
    ins = (u_in, i_in)
    outs = (u_out, i_out)

    def gather(ln, it):
        t, offs = _LANES[ln]
        return pltpu.make_async_copy(
            ins[t].at[pl.ds(offs[it], _CHUNK)], bufs[ln], sems[ln]
        )

    def scatter(ln, it):
        t, offs = _LANES[ln]
        return pltpu.make_async_copy(
            bufs[ln], outs[t].at[pl.ds(offs[it], _CHUNK)], sems[ln]
        )

    for ln in range(len(_LANES)):
        gather(ln, 0).start()
    for it in range(_MAX_ITERS):
        couts = {}
        for ln, (t, offs) in enumerate(_LANES):
            if it < len(offs):
                gather(ln, it).wait()
                c = scatter(ln, it)
                c.start()
                couts[ln] = c
        for ln, (t, offs) in enumerate(_LANES):
            if ln in couts:
                couts[ln].wait()
                if it + 1 < len(offs):
                    gather(ln, it + 1).start()


def kernel(user_weight, item_weight):
    n = len(_LANES)
    return pl.pallas_call(
        _copy_kernel,
        out_shape=(
            jax.ShapeDtypeStruct(user_weight.shape, user_weight.dtype),
            jax.ShapeDtypeStruct(item_weight.shape, item_weight.dtype),
        ),
        in_specs=[
            pl.BlockSpec(memory_space=pltpu.MemorySpace.HBM),
            pl.BlockSpec(memory_space=pltpu.MemorySpace.HBM),
        ],
        out_specs=(
            pl.BlockSpec(memory_space=pltpu.MemorySpace.HBM),
            pl.BlockSpec(memory_space=pltpu.MemorySpace.HBM),
        ),
        scratch_shapes=(
            [pltpu.VMEM((_CHUNK, _EMBED), jnp.float32) for _ in range(n)]
            + [pltpu.SemaphoreType.DMA for _ in range(n)]
        ),
    )(user_weight, item_weight)
